# Initial kernel scaffold; baseline (speedup 1.0000x reference)
import jax
import jax.numpy as jnp
from jax.experimental import pallas as pl


def kernel(x, edge_index, W1, b1, W2, b2, W3, b3):
    raise NotImplementedError("write your pallas kernel here")


# speedup vs baseline: 122.5275x; 122.5275x over previous
import functools

import jax
import jax.numpy as jnp
from jax import lax
from jax.experimental import pallas as pl
from jax.experimental.pallas import tpu as pltpu
from jax.experimental.pallas import tpu_sc as plsc

N = 100000
E = 6400000
NC, NS = 2, 16
NW = NC * NS
NP = 100352
PW = NP // NW
SW = NP // NS
EW = E // NW
C = 2000
K = EW // C
FP = 8

_CP = pltpu.CompilerParams(use_tc_tiling_on_sc=False, needs_layout_passes=False)


def _mesh():
  return plsc.VectorSubcoreMesh(core_axis_name="c", subcore_axis_name="s")


def _wid():
  return lax.axis_index("s") * NC + lax.axis_index("c")


def _rsqrt(d):
  y = plsc.bitcast(
      jnp.int32(0x5F3759DF)
      - lax.shift_right_logical(plsc.bitcast(d, jnp.int32), 1),
      jnp.float32,
  )
  for _ in range(3):
    y = y * (1.5 - 0.5 * d * y * y)
  return y


def _tanh(o):
  e = jnp.exp(2.0 * o)
  return 1.0 - 2.0 / (e + 1.0)


@functools.lru_cache(None)
def _edge_call():
  @functools.partial(
      pl.kernel,
      out_type=jax.ShapeDtypeStruct((NC, NP, FP), jnp.float32),
      mesh=_mesh(),
      compiler_params=_CP,
      scratch_types=[
          pltpu.VMEM_SHARED((NP, FP), jnp.float32),
          pltpu.VMEM_SHARED((NP, FP), jnp.float32),
          pltpu.VMEM((C,), jnp.int32),
          pltpu.VMEM((C,), jnp.int32),
          pltpu.VMEM((C, FP), jnp.float32),
          pltpu.SemaphoreType.DMA,
      ],
  )
  def edge_kernel(u_hbm, src_hbm, dst_hbm, z_hbm, out_hbm,
                  ush, accsh, sbuf, dbuf, rows, sem):
    c = lax.axis_index("c")
    s = lax.axis_index("s")
    w = s * NC + c
    r0 = s * SW
    pltpu.sync_copy(u_hbm.at[pl.ds(r0, SW)], ush.at[pl.ds(r0, SW)])
    pltpu.sync_copy(z_hbm.at[pl.ds(r0, SW)], accsh.at[pl.ds(r0, SW)])
    plsc.subcore_barrier()
    base = w * EW

    @pl.loop(0, K)
    def _chunk(k):
      off = base + k * C
      pltpu.sync_copy(src_hbm.at[pl.ds(off, C)], sbuf)
      pltpu.sync_copy(dst_hbm.at[pl.ds(off, C)], dbuf)
      pltpu.async_copy(ush.at[sbuf], rows, sem).wait()
      pltpu.sync_copy(rows, accsh.at[dbuf], add=True)

    plsc.subcore_barrier()
    pltpu.sync_copy(accsh.at[pl.ds(r0, SW)], out_hbm.at[c, pl.ds(r0, SW)])

  return edge_kernel


@functools.lru_cache(None)
def _glue0_call():
  @functools.partial(
      pl.kernel,
      out_type=(jax.ShapeDtypeStruct((NP,), jnp.float32),
                jax.ShapeDtypeStruct((NP * FP,), jnp.float32)),
      mesh=_mesh(),
      compiler_params=_CP,
      scratch_types=[
          pltpu.VMEM((FP * PW,), jnp.float32),
          pltpu.VMEM((FP * PW,), jnp.float32),
          pltpu.VMEM((2 * PW,), jnp.float32),
          pltpu.VMEM((PW,), jnp.float32),
          pltpu.VMEM((FP * PW,), jnp.float32),
          pltpu.VMEM((16,), jnp.float32),
      ],
  )
  def glue0(part_hbm, x_hbm, w_hbm, dis_hbm, u_hbm, p0b, p1b, xb, db, ub, wv):
    w = _wid()
    nb = w * PW
    pltpu.sync_copy(part_hbm.at[0, pl.ds(FP * nb, FP * PW)], p0b)
    pltpu.sync_copy(part_hbm.at[1, pl.ds(FP * nb, FP * PW)], p1b)
    pltpu.sync_copy(x_hbm.at[pl.ds(2 * nb, 2 * PW)], xb)
    pltpu.sync_copy(w_hbm, wv)
    iota = lax.iota(jnp.int32, 16)
    col = lax.bitwise_and(iota, FP - 1)
    w0t = plsc.load_gather(wv, [col])
    w1t = plsc.load_gather(wv, [col + FP])

    @pl.loop(0, PW // 16)
    def _i(i):
      d = (plsc.load_gather(p0b, [i * 128 + iota * FP])
           + plsc.load_gather(p1b, [i * 128 + iota * FP]) + 1.0)
      db[pl.ds(i * 16, 16)] = _rsqrt(d)

    @pl.loop(0, PW // 2)
    def _i2(i):
      node = i * 2 + lax.shift_right_logical(iota, 3)
      d = (plsc.load_gather(p0b, [node * FP])
           + plsc.load_gather(p1b, [node * FP]) + 1.0)
      r = _rsqrt(d)
      x0 = plsc.load_gather(xb, [node * 2])
      x1 = plsc.load_gather(xb, [node * 2 + 1])
      ub[pl.ds(i * 16, 16)] = r * (x0 * w0t + x1 * w1t)

    pltpu.sync_copy(db, dis_hbm.at[pl.ds(nb, PW)])
    pltpu.sync_copy(ub, u_hbm.at[pl.ds(FP * nb, FP * PW)])

  return glue0


@functools.lru_cache(None)
def _glueN_call(Fout):
  Fin = 4

  @functools.partial(
      pl.kernel,
      out_type=jax.ShapeDtypeStruct((NP * FP,), jnp.float32),
      mesh=_mesh(),
      compiler_params=_CP,
      scratch_types=[
          pltpu.VMEM((FP * PW,), jnp.float32),
          pltpu.VMEM((FP * PW,), jnp.float32),
          pltpu.VMEM((FP * PW,), jnp.float32),
          pltpu.VMEM((PW,), jnp.float32),
          pltpu.VMEM((32,), jnp.float32),
          pltpu.VMEM((16,), jnp.float32),
      ],
  )
  def glueN(part_hbm, u_hbm, dis_hbm, w_hbm, b_hbm, un_hbm,
            p0b, p1b, ub, db, wv, bv):
    w = _wid()
    nb = w * PW
    fb = FP * nb
    pltpu.sync_copy(part_hbm.at[0, pl.ds(fb, FP * PW)], p0b)
    pltpu.sync_copy(part_hbm.at[1, pl.ds(fb, FP * PW)], p1b)
    pltpu.sync_copy(u_hbm.at[pl.ds(fb, FP * PW)], ub)
    pltpu.sync_copy(dis_hbm.at[pl.ds(nb, PW)], db)
    pltpu.sync_copy(w_hbm, wv)
    pltpu.sync_copy(b_hbm, bv)
    iota = lax.iota(jnp.int32, 16)
    col = lax.bitwise_and(iota, FP - 1)
    lane_node = lax.shift_right_logical(iota, 3)
    bt = plsc.load_gather(bv, [col])
    wkt = [plsc.load_gather(wv, [k * FP + col]) for k in range(Fin)]

    @pl.loop(0, PW // 2)
    def _i(i):
      sl = pl.ds(i * 16, 16)
      s16 = p0b[sl] + p1b[sl] + ub[sl]
      d2 = plsc.load_gather(db, [i * 2 + lane_node])
      h16 = _tanh(d2 * s16 + bt)
      acc = jnp.zeros((16,), jnp.float32)
      for k in range(Fin):
        hk = jnp.take_along_axis(h16, lane_node * FP + k, axis=0)
        acc = acc + hk * wkt[k]
      ub[sl] = d2 * acc

    pltpu.sync_copy(ub, un_hbm.at[pl.ds(fb, FP * PW)])

  return glueN


@functools.lru_cache(None)
def _glueF_call():
  Fo = 2

  @functools.partial(
      pl.kernel,
      out_type=jax.ShapeDtypeStruct((NP * Fo,), jnp.float32),
      mesh=_mesh(),
      compiler_params=_CP,
      scratch_types=[
          pltpu.VMEM((FP * PW,), jnp.float32),
          pltpu.VMEM((FP * PW,), jnp.float32),
          pltpu.VMEM((FP * PW,), jnp.float32),
          pltpu.VMEM((PW,), jnp.float32),
          pltpu.VMEM((Fo * PW,), jnp.float32),
          pltpu.VMEM((16,), jnp.float32),
      ],
  )
  def glueF(part_hbm, u_hbm, dis_hbm, b_hbm, o_hbm, p0b, p1b, ub, db, ob, bv):
    w = _wid()
    nb = w * PW
    fb = FP * nb
    pltpu.sync_copy(part_hbm.at[0, pl.ds(fb, FP * PW)], p0b)
    pltpu.sync_copy(part_hbm.at[1, pl.ds(fb, FP * PW)], p1b)
    pltpu.sync_copy(u_hbm.at[pl.ds(fb, FP * PW)], ub)
    pltpu.sync_copy(dis_hbm.at[pl.ds(nb, PW)], db)
    pltpu.sync_copy(b_hbm, bv)
    iota = lax.iota(jnp.int32, 16)
    bt = plsc.load_gather(bv, [lax.bitwise_and(iota, Fo - 1)])

    @pl.loop(0, Fo * PW // 16)
    def _i(i):
      fidx = ((i * 8 + lax.shift_right_logical(iota, 1)) * FP
              + lax.bitwise_and(iota, Fo - 1))
      s16 = (plsc.load_gather(p0b, [fidx]) + plsc.load_gather(p1b, [fidx])
             + plsc.load_gather(ub, [fidx]))
      d2 = plsc.load_gather(db, [i * 8 + lax.shift_right_logical(iota, 1)])
      ob[pl.ds(i * 16, 16)] = d2 * s16 + bt

    pltpu.sync_copy(ob, o_hbm.at[pl.ds(Fo * nb, Fo * PW)])

  return glueF


def _pad16(a):
  return jnp.zeros((16,), jnp.float32).at[: a.size].set(
      a.reshape(-1).astype(jnp.float32))


def _pad_w(a, rows):
  out = jnp.zeros((rows, FP), jnp.float32)
  return out.at[: a.shape[0], : a.shape[1]].set(
      a.astype(jnp.float32)).reshape(-1)


def kernel(x, edge_index, W1, b1, W2, b2, W3, b3):
  f32 = jnp.float32
  src = edge_index[0].astype(jnp.int32)
  dst = edge_index[1].astype(jnp.int32)
  xp = jnp.zeros((NP, 2), f32).at[:N].set(x.astype(f32)).reshape(-1)
  ones8 = jnp.ones((NP, FP), f32)
  z8 = jnp.zeros((NP, FP), f32)
  w1p = _pad_w(W1, 2)
  w2p, w3p = _pad_w(W2, 4), _pad_w(W3, 4)
  b1p, b2p, b3p = _pad16(b1), _pad16(b2), _pad16(b3)

  edge = _edge_call()
  degp = edge(ones8, src, dst, z8)
  dis, u1 = _glue0_call()(degp.reshape(NC, NP * FP), xp, w1p)
  p1_ = edge(u1.reshape(NP, FP), src, dst, z8)
  u2 = _glueN_call(4)(p1_.reshape(NC, NP * FP), u1, dis, w2p, b1p)
  p2_ = edge(u2.reshape(NP, FP), src, dst, z8)
  u3 = _glueN_call(2)(p2_.reshape(NC, NP * FP), u2, dis, w3p, b2p)
  p3_ = edge(u3.reshape(NP, FP), src, dst, z8)
  out = _glueF_call()(p3_.reshape(NC, NP * FP), u3, dis, b3p)
  return out.reshape(NP, 2)[:N]


# speedup vs baseline: 127.1870x; 1.0380x over previous
import functools

import jax
import jax.numpy as jnp
from jax import lax
from jax.experimental import pallas as pl
from jax.experimental.pallas import tpu as pltpu
from jax.experimental.pallas import tpu_sc as plsc

N = 100000
E = 6400000
NC, NS = 2, 16
NW = NC * NS
NP = 100352
PW = NP // NW
SW = NP // NS
EW = E // NW
C = 1000
K = EW // C
FP = 8

_CP = pltpu.CompilerParams(use_tc_tiling_on_sc=False, needs_layout_passes=False)


def _mesh():
  return plsc.VectorSubcoreMesh(core_axis_name="c", subcore_axis_name="s")


def _wid():
  return lax.axis_index("s") * NC + lax.axis_index("c")


def _rsqrt(d):
  y = plsc.bitcast(
      jnp.int32(0x5F3759DF)
      - lax.shift_right_logical(plsc.bitcast(d, jnp.int32), 1),
      jnp.float32,
  )
  for _ in range(3):
    y = y * (1.5 - 0.5 * d * y * y)
  return y


def _tanh(o):
  e = jnp.exp(2.0 * o)
  return 1.0 - 2.0 / (e + 1.0)


@functools.lru_cache(None)
def _edge_call():
  @functools.partial(
      pl.kernel,
      out_type=jax.ShapeDtypeStruct((NC, NP, FP), jnp.float32),
      mesh=_mesh(),
      compiler_params=_CP,
      scratch_types=[
          pltpu.VMEM_SHARED((NP, FP), jnp.float32),
          pltpu.VMEM_SHARED((NP, FP), jnp.float32),
          pltpu.VMEM((C,), jnp.int32),
          pltpu.VMEM((C,), jnp.int32),
          pltpu.VMEM((C,), jnp.int32),
          pltpu.VMEM((C,), jnp.int32),
          pltpu.VMEM((C, FP), jnp.float32),
          pltpu.VMEM((C, FP), jnp.float32),
          pltpu.SemaphoreType.DMA,
          pltpu.SemaphoreType.DMA,
          pltpu.SemaphoreType.DMA,
      ],
  )
  def edge_kernel(u_hbm, src_hbm, dst_hbm, z_hbm, out_hbm,
                  ush, accsh, sbuf0, sbuf1, dbuf0, dbuf1, rows0, rows1,
                  gsem, ssem0, ssem1):
    c = lax.axis_index("c")
    s = lax.axis_index("s")
    w = s * NC + c
    r0 = s * SW
    pltpu.sync_copy(u_hbm.at[pl.ds(r0, SW)], ush.at[pl.ds(r0, SW)])
    pltpu.sync_copy(z_hbm.at[pl.ds(r0, SW)], accsh.at[pl.ds(r0, SW)])
    plsc.subcore_barrier()
    base = w * EW
    bufs = ((sbuf0, dbuf0, rows0, ssem0), (sbuf1, dbuf1, rows1, ssem1))

    @pl.loop(0, K, step=2)
    def _chunk(k):
      for b, (sb, db, rows, ssem) in enumerate(bufs):
        @pl.when(k >= 2)
        def _():
          pltpu.make_async_copy(rows, accsh.at[db], ssem).wait()
        off = base + (k + b) * C
        pltpu.sync_copy(src_hbm.at[pl.ds(off, C)], sb)
        pltpu.sync_copy(dst_hbm.at[pl.ds(off, C)], db)
        pltpu.async_copy(ush.at[sb], rows, gsem).wait()
        pltpu.async_copy(rows, accsh.at[db], ssem, add=True)

    for (sb, db, rows, ssem) in bufs:
      pltpu.make_async_copy(rows, accsh.at[db], ssem).wait()
    plsc.subcore_barrier()
    pltpu.sync_copy(accsh.at[pl.ds(r0, SW)], out_hbm.at[c, pl.ds(r0, SW)])

  return edge_kernel


@functools.lru_cache(None)
def _deg_call():
  @functools.partial(
      pl.kernel,
      out_type=jax.ShapeDtypeStruct((NC, NP, FP), jnp.float32),
      mesh=_mesh(),
      compiler_params=_CP,
      scratch_types=[
          pltpu.VMEM_SHARED((NP, FP), jnp.float32),
          pltpu.VMEM((C,), jnp.int32),
          pltpu.VMEM((C,), jnp.int32),
          pltpu.VMEM((C, FP), jnp.float32),
          pltpu.SemaphoreType.DMA,
          pltpu.SemaphoreType.DMA,
      ],
  )
  def deg_kernel(ones_hbm, dst_hbm, z_hbm, out_hbm,
                 accsh, dbuf0, dbuf1, ones, ssem0, ssem1):
    c = lax.axis_index("c")
    s = lax.axis_index("s")
    w = s * NC + c
    r0 = s * SW
    pltpu.sync_copy(z_hbm.at[pl.ds(r0, SW)], accsh.at[pl.ds(r0, SW)])
    pltpu.sync_copy(ones_hbm.at[pl.ds(0, C)], ones)
    plsc.subcore_barrier()
    base = w * EW
    bufs = ((dbuf0, ssem0), (dbuf1, ssem1))

    @pl.loop(0, K, step=2)
    def _chunk(k):
      for b, (db, ssem) in enumerate(bufs):
        @pl.when(k >= 2)
        def _():
          pltpu.make_async_copy(ones, accsh.at[db], ssem).wait()
        off = base + (k + b) * C
        pltpu.sync_copy(dst_hbm.at[pl.ds(off, C)], db)
        pltpu.async_copy(ones, accsh.at[db], ssem, add=True)

    for (db, ssem) in bufs:
      pltpu.make_async_copy(ones, accsh.at[db], ssem).wait()
    plsc.subcore_barrier()
    pltpu.sync_copy(accsh.at[pl.ds(r0, SW)], out_hbm.at[c, pl.ds(r0, SW)])

  return deg_kernel


@functools.lru_cache(None)
def _glue0_call():
  @functools.partial(
      pl.kernel,
      out_type=(jax.ShapeDtypeStruct((NP,), jnp.float32),
                jax.ShapeDtypeStruct((NP * FP,), jnp.float32)),
      mesh=_mesh(),
      compiler_params=_CP,
      scratch_types=[
          pltpu.VMEM((FP * PW,), jnp.float32),
          pltpu.VMEM((FP * PW,), jnp.float32),
          pltpu.VMEM((2 * PW,), jnp.float32),
          pltpu.VMEM((PW,), jnp.float32),
          pltpu.VMEM((FP * PW,), jnp.float32),
          pltpu.VMEM((16,), jnp.float32),
      ],
  )
  def glue0(part_hbm, x_hbm, w_hbm, dis_hbm, u_hbm, p0b, p1b, xb, db, ub, wv):
    w = _wid()
    nb = w * PW
    pltpu.sync_copy(part_hbm.at[0, pl.ds(FP * nb, FP * PW)], p0b)
    pltpu.sync_copy(part_hbm.at[1, pl.ds(FP * nb, FP * PW)], p1b)
    pltpu.sync_copy(x_hbm.at[pl.ds(2 * nb, 2 * PW)], xb)
    pltpu.sync_copy(w_hbm, wv)
    iota = lax.iota(jnp.int32, 16)
    col = lax.bitwise_and(iota, FP - 1)
    w0t = plsc.load_gather(wv, [col])
    w1t = plsc.load_gather(wv, [col + FP])

    @pl.loop(0, PW // 16)
    def _i(i):
      d = (plsc.load_gather(p0b, [i * 128 + iota * FP])
           + plsc.load_gather(p1b, [i * 128 + iota * FP]) + 1.0)
      db[pl.ds(i * 16, 16)] = _rsqrt(d)

    @pl.loop(0, PW // 2)
    def _i2(i):
      node = i * 2 + lax.shift_right_logical(iota, 3)
      d = (plsc.load_gather(p0b, [node * FP])
           + plsc.load_gather(p1b, [node * FP]) + 1.0)
      r = _rsqrt(d)
      x0 = plsc.load_gather(xb, [node * 2])
      x1 = plsc.load_gather(xb, [node * 2 + 1])
      ub[pl.ds(i * 16, 16)] = r * (x0 * w0t + x1 * w1t)

    pltpu.sync_copy(db, dis_hbm.at[pl.ds(nb, PW)])
    pltpu.sync_copy(ub, u_hbm.at[pl.ds(FP * nb, FP * PW)])

  return glue0


@functools.lru_cache(None)
def _glueN_call(Fout):
  Fin = 4

  @functools.partial(
      pl.kernel,
      out_type=jax.ShapeDtypeStruct((NP * FP,), jnp.float32),
      mesh=_mesh(),
      compiler_params=_CP,
      scratch_types=[
          pltpu.VMEM((FP * PW,), jnp.float32),
          pltpu.VMEM((FP * PW,), jnp.float32),
          pltpu.VMEM((FP * PW,), jnp.float32),
          pltpu.VMEM((PW,), jnp.float32),
          pltpu.VMEM((32,), jnp.float32),
          pltpu.VMEM((16,), jnp.float32),
      ],
  )
  def glueN(part_hbm, u_hbm, dis_hbm, w_hbm, b_hbm, un_hbm,
            p0b, p1b, ub, db, wv, bv):
    w = _wid()
    nb = w * PW
    fb = FP * nb
    pltpu.sync_copy(part_hbm.at[0, pl.ds(fb, FP * PW)], p0b)
    pltpu.sync_copy(part_hbm.at[1, pl.ds(fb, FP * PW)], p1b)
    pltpu.sync_copy(u_hbm.at[pl.ds(fb, FP * PW)], ub)
    pltpu.sync_copy(dis_hbm.at[pl.ds(nb, PW)], db)
    pltpu.sync_copy(w_hbm, wv)
    pltpu.sync_copy(b_hbm, bv)
    iota = lax.iota(jnp.int32, 16)
    col = lax.bitwise_and(iota, FP - 1)
    lane_node = lax.shift_right_logical(iota, 3)
    bt = plsc.load_gather(bv, [col])
    wkt = [plsc.load_gather(wv, [k * FP + col]) for k in range(Fin)]

    @pl.loop(0, PW // 2)
    def _i(i):
      sl = pl.ds(i * 16, 16)
      s16 = p0b[sl] + p1b[sl] + ub[sl]
      d2 = plsc.load_gather(db, [i * 2 + lane_node])
      h16 = _tanh(d2 * s16 + bt)
      acc = jnp.zeros((16,), jnp.float32)
      for k in range(Fin):
        hk = jnp.take_along_axis(h16, lane_node * FP + k, axis=0)
        acc = acc + hk * wkt[k]
      ub[sl] = d2 * acc

    pltpu.sync_copy(ub, un_hbm.at[pl.ds(fb, FP * PW)])

  return glueN


@functools.lru_cache(None)
def _glueF_call():
  Fo = 2

  @functools.partial(
      pl.kernel,
      out_type=jax.ShapeDtypeStruct((NP * Fo,), jnp.float32),
      mesh=_mesh(),
      compiler_params=_CP,
      scratch_types=[
          pltpu.VMEM((FP * PW,), jnp.float32),
          pltpu.VMEM((FP * PW,), jnp.float32),
          pltpu.VMEM((FP * PW,), jnp.float32),
          pltpu.VMEM((PW,), jnp.float32),
          pltpu.VMEM((Fo * PW,), jnp.float32),
          pltpu.VMEM((16,), jnp.float32),
      ],
  )
  def glueF(part_hbm, u_hbm, dis_hbm, b_hbm, o_hbm, p0b, p1b, ub, db, ob, bv):
    w = _wid()
    nb = w * PW
    fb = FP * nb
    pltpu.sync_copy(part_hbm.at[0, pl.ds(fb, FP * PW)], p0b)
    pltpu.sync_copy(part_hbm.at[1, pl.ds(fb, FP * PW)], p1b)
    pltpu.sync_copy(u_hbm.at[pl.ds(fb, FP * PW)], ub)
    pltpu.sync_copy(dis_hbm.at[pl.ds(nb, PW)], db)
    pltpu.sync_copy(b_hbm, bv)
    iota = lax.iota(jnp.int32, 16)
    bt = plsc.load_gather(bv, [lax.bitwise_and(iota, Fo - 1)])

    @pl.loop(0, Fo * PW // 16)
    def _i(i):
      fidx = ((i * 8 + lax.shift_right_logical(iota, 1)) * FP
              + lax.bitwise_and(iota, Fo - 1))
      s16 = (plsc.load_gather(p0b, [fidx]) + plsc.load_gather(p1b, [fidx])
             + plsc.load_gather(ub, [fidx]))
      d2 = plsc.load_gather(db, [i * 8 + lax.shift_right_logical(iota, 1)])
      ob[pl.ds(i * 16, 16)] = d2 * s16 + bt

    pltpu.sync_copy(ob, o_hbm.at[pl.ds(Fo * nb, Fo * PW)])

  return glueF


def _pad16(a):
  return jnp.zeros((16,), jnp.float32).at[: a.size].set(
      a.reshape(-1).astype(jnp.float32))


def _pad_w(a, rows):
  out = jnp.zeros((rows, FP), jnp.float32)
  return out.at[: a.shape[0], : a.shape[1]].set(
      a.astype(jnp.float32)).reshape(-1)


def kernel(x, edge_index, W1, b1, W2, b2, W3, b3):
  f32 = jnp.float32
  src = edge_index[0].astype(jnp.int32)
  dst = edge_index[1].astype(jnp.int32)
  xp = jnp.zeros((NP, 2), f32).at[:N].set(x.astype(f32)).reshape(-1)
  ones8 = jnp.ones((NP, FP), f32)
  z8 = jnp.zeros((NP, FP), f32)
  w1p = _pad_w(W1, 2)
  w2p, w3p = _pad_w(W2, 4), _pad_w(W3, 4)
  b1p, b2p, b3p = _pad16(b1), _pad16(b2), _pad16(b3)

  edge = _edge_call()
  degp = _deg_call()(ones8, dst, z8)
  dis, u1 = _glue0_call()(degp.reshape(NC, NP * FP), xp, w1p)
  p1_ = edge(u1.reshape(NP, FP), src, dst, z8)
  u2 = _glueN_call(4)(p1_.reshape(NC, NP * FP), u1, dis, w2p, b1p)
  p2_ = edge(u2.reshape(NP, FP), src, dst, z8)
  u3 = _glueN_call(2)(p2_.reshape(NC, NP * FP), u2, dis, w3p, b2p)
  p3_ = edge(u3.reshape(NP, FP), src, dst, z8)
  out = _glueF_call()(p3_.reshape(NC, NP * FP), u3, dis, b3p)
  return out.reshape(NP, 2)[:N]
